# Initial kernel scaffold; baseline (speedup 1.0000x reference)
import functools

import jax
import jax.numpy as jnp
from jax import lax
from jax.experimental import pallas as pl
from jax.experimental.pallas import tpu as pltpu
from jax.experimental.pallas import tpu_sc as plsc

N = 10000
E = 320000
Q = 320000
D = 128
H = 128

NC = 2
NS = 16
NW = NC * NS
RPW = 320
NPAD = NW * RPW
NR = 336
DUMMY = 320
C = 2000
NCHUNK = E // C

QPW = Q // NW
CG = 128
NGC = 79


def _mesh():
    return plsc.VectorSubcoreMesh(core_axis_name="c", subcore_axis_name="s")


def _agg_body(h_hbm, src_hbm, dst_hbm, sum_hbm, cnt_hbm, max_hbm,
              asum, amax, acnt, dstb, srcb, cidx, cld, rows, sem):
    wid = lax.axis_index("s") * NC + lax.axis_index("c")
    lo = wid * RPW
    hi = lo + RPW

    zero16 = jnp.zeros((16,), jnp.float32)
    ninf16 = jnp.full((16,), -jnp.inf, jnp.float32)

    @pl.loop(0, NR)
    def _init(r):
        for kk in range(8):
            sl = pl.ds(16 * kk, 16)
            asum[r, sl] = zero16
            amax[r, sl] = ninf16

    @pl.loop(0, NR // 16)
    def _initc(r):
        acnt[pl.ds(16 * r, 16)] = zero16

    @pl.loop(0, NCHUNK)
    def _chunk(c):
        off = pl.multiple_of(c * C, 8)
        pltpu.sync_copy(src_hbm.at[pl.ds(off, C)], srcb)
        pltpu.sync_copy(dst_hbm.at[pl.ds(off, C)], dstb)

        @pl.loop(0, C // 16, init_carry=0)
        def _scan(i, n):
            d = dstb[pl.ds(16 * i, 16)]
            s = srcb[pl.ds(16 * i, 16)]
            m = (d >= lo) & (d < hi)
            plsc.store_compressed(cidx.at[pl.ds(n, 16)], s, mask=m)
            plsc.store_compressed(cld.at[pl.ds(n, 16)], d - lo, mask=m)
            return n + jnp.max(plsc.all_reduce_population_count(m))

        n = _scan
        cidx[pl.ds(n, 16)] = jnp.zeros((16,), jnp.int32)
        cld[pl.ds(n, 16)] = jnp.full((16,), DUMMY, jnp.int32)
        nb = (n + 15) // 16

        @pl.loop(0, nb)
        def _gblk(g):
            gi = cidx[pl.ds(16 * g, 16)]
            pltpu.async_copy(h_hbm.at[gi], rows, sem).wait()
            ld16 = cld[pl.ds(16 * g, 16)]

            @pl.loop(0, 16)
            def _acc(j):
                ld = cld[16 * g + j]
                for kk in range(8):
                    sl = pl.ds(16 * kk, 16)
                    r = rows[j, sl]
                    asum[ld, sl] = asum[ld, sl] + r
                    amax[ld, sl] = jnp.maximum(amax[ld, sl], r)

            plsc.addupdate_scatter(acnt, [ld16], jnp.ones((16,), jnp.float32))

    lo8 = pl.multiple_of(lo, 8)
    pltpu.sync_copy(asum.at[pl.ds(0, RPW)], sum_hbm.at[pl.ds(lo8, RPW)])
    pltpu.sync_copy(amax.at[pl.ds(0, RPW)], max_hbm.at[pl.ds(lo8, RPW)])
    pltpu.sync_copy(acnt.at[pl.ds(0, RPW)], cnt_hbm.at[pl.ds(lo8, RPW)])


@jax.jit
def _sc_agg(h, src, dst):
    f32 = jnp.float32
    return pl.kernel(
        _agg_body,
        out_type=(
            jax.ShapeDtypeStruct((NPAD, H), f32),
            jax.ShapeDtypeStruct((NPAD,), f32),
            jax.ShapeDtypeStruct((NPAD, H), f32),
        ),
        mesh=_mesh(),
        scratch_types=[
            pltpu.VMEM((NR, H), f32),
            pltpu.VMEM((NR, H), f32),
            pltpu.VMEM((NR,), f32),
            pltpu.VMEM((C,), jnp.int32),
            pltpu.VMEM((C,), jnp.int32),
            pltpu.VMEM((C + 16,), jnp.int32),
            pltpu.VMEM((C + 16,), jnp.int32),
            pltpu.VMEM((16, H), f32),
            pltpu.SemaphoreType.DMA,
        ],
    )(h, src, dst)


def _gather_body(tab_hbm, idx_hbm, out_hbm, idxb, rowsb, sem):
    wid = lax.axis_index("s") * NC + lax.axis_index("c")
    base = wid * QPW

    @pl.loop(0, NGC)
    def _t(t):
        off = jnp.minimum(base + t * CG, Q - CG)
        off = pl.multiple_of(off, 8)
        pltpu.sync_copy(idx_hbm.at[pl.ds(off, CG)], idxb)
        pltpu.async_copy(tab_hbm.at[idxb], rowsb, sem).wait()
        pltpu.sync_copy(rowsb, out_hbm.at[pl.ds(off, CG)])


@jax.jit
def _sc_gather(tab, idx):
    return pl.kernel(
        _gather_body,
        out_type=jax.ShapeDtypeStruct((Q, H), jnp.float32),
        mesh=_mesh(),
        scratch_types=[
            pltpu.VMEM((CG,), jnp.int32),
            pltpu.VMEM((CG, H), jnp.float32),
            pltpu.SemaphoreType.DMA,
        ],
    )(tab, idx)


def _enc_body(x_ref, w_ref, b_ref, o_ref):
    o_ref[...] = (
        jnp.dot(x_ref[...], w_ref[...], preferred_element_type=jnp.float32)
        + b_ref[...]
    )


@jax.jit
def _tc_enc(x, w, b):
    BN = 1000
    return pl.pallas_call(
        _enc_body,
        grid=(N // BN,),
        in_specs=[
            pl.BlockSpec((BN, D), lambda i: (i, 0)),
            pl.BlockSpec((D, H), lambda i: (0, 0)),
            pl.BlockSpec((H,), lambda i: (0,)),
        ],
        out_specs=pl.BlockSpec((BN, H), lambda i: (i, 0)),
        out_shape=jax.ShapeDtypeStruct((N, H), jnp.float32),
    )(x, w, b)


def _layer_body(h_ref, s_ref, c_ref, m_ref, wl_ref, bl_ref, wr_ref, g_ref, b_ref, o_ref):
    h = h_ref[...]
    cnt = c_ref[...]
    mean = s_ref[...] / jnp.maximum(cnt, 1.0)
    mx = m_ref[...]
    mx = jnp.where(jnp.isfinite(mx), mx, 0.0)
    agg = jnp.concatenate([mean, mx], axis=1)
    out = (
        jnp.dot(agg, wl_ref[...], preferred_element_type=jnp.float32)
        + bl_ref[...]
        + jnp.dot(h, wr_ref[...], preferred_element_type=jnp.float32)
    )
    mu = jnp.mean(out, axis=-1, keepdims=True)
    v = jnp.mean((out - mu) ** 2, axis=-1, keepdims=True)
    out = (out - mu) / jnp.sqrt(v + 1e-5) * g_ref[...] + b_ref[...]
    o_ref[...] = jnp.maximum(out, 0.0) + h


@jax.jit
def _tc_layer(h, s, cnt2, mx, wl, bl, wr, g, b):
    BN = 1000
    return pl.pallas_call(
        _layer_body,
        grid=(N // BN,),
        in_specs=[
            pl.BlockSpec((BN, H), lambda i: (i, 0)),
            pl.BlockSpec((BN, H), lambda i: (i, 0)),
            pl.BlockSpec((BN, 1), lambda i: (i, 0)),
            pl.BlockSpec((BN, H), lambda i: (i, 0)),
            pl.BlockSpec((2 * H, H), lambda i: (0, 0)),
            pl.BlockSpec((H,), lambda i: (0,)),
            pl.BlockSpec((H, H), lambda i: (0, 0)),
            pl.BlockSpec((H,), lambda i: (0,)),
            pl.BlockSpec((H,), lambda i: (0,)),
        ],
        out_specs=pl.BlockSpec((BN, H), lambda i: (i, 0)),
        out_shape=jax.ShapeDtypeStruct((N, H), jnp.float32),
    )(h, s, cnt2, mx, wl, bl, wr, g, b)


def _mlp_body(hs_ref, ht_ref, a_ref, w1a_ref, w1b_ref, w1c_ref, b1_ref,
              g_ref, bb_ref, w2_ref, b2_ref, w3_ref, b3_ref, o_ref):
    z = (
        jnp.dot(hs_ref[...], w1a_ref[...], preferred_element_type=jnp.float32)
        + jnp.dot(ht_ref[...], w1b_ref[...], preferred_element_type=jnp.float32)
        + jnp.dot(a_ref[...], w1c_ref[...], preferred_element_type=jnp.float32)
        + b1_ref[...]
    )
    z = z * g_ref[...] + bb_ref[...]
    z = jnp.maximum(z, 0.0)
    z = jnp.maximum(
        jnp.dot(z, w2_ref[...], preferred_element_type=jnp.float32) + b2_ref[...],
        0.0,
    )
    o_ref[...] = (
        jnp.dot(z, w3_ref[...], preferred_element_type=jnp.float32) + b3_ref[...]
    )


@jax.jit
def _tc_mlp(hs, ht, attr, w1a, w1b, w1c, b1, g, bb, w2, b2, w3, b3):
    BQ = 3200
    DE = attr.shape[1]
    return pl.pallas_call(
        _mlp_body,
        grid=(Q // BQ,),
        in_specs=[
            pl.BlockSpec((BQ, H), lambda i: (i, 0)),
            pl.BlockSpec((BQ, H), lambda i: (i, 0)),
            pl.BlockSpec((BQ, DE), lambda i: (i, 0)),
            pl.BlockSpec((H, 2 * H), lambda i: (0, 0)),
            pl.BlockSpec((H, 2 * H), lambda i: (0, 0)),
            pl.BlockSpec((DE, 2 * H), lambda i: (0, 0)),
            pl.BlockSpec((2 * H,), lambda i: (0,)),
            pl.BlockSpec((2 * H,), lambda i: (0,)),
            pl.BlockSpec((2 * H,), lambda i: (0,)),
            pl.BlockSpec((2 * H, H), lambda i: (0, 0)),
            pl.BlockSpec((H,), lambda i: (0,)),
            pl.BlockSpec((H, 1), lambda i: (0, 0)),
            pl.BlockSpec((1,), lambda i: (0,)),
        ],
        out_specs=pl.BlockSpec((BQ, 1), lambda i: (i, 0)),
        out_shape=jax.ShapeDtypeStruct((Q, 1), jnp.float32),
    )(hs, ht, attr, w1a, w1b, w1c, b1, g, bb, w2, b2, w3, b3)


def kernel(x, edge_index, edge_attr, query_edge_indices, W_enc, b_enc,
           W_l0, b_l0, W_r0, ln_g0, ln_b0, W_l1, b_l1, W_r1, ln_g1, ln_b1,
           W_l2, b_l2, W_r2, ln_g2, ln_b2, W_e1, b_e1, bn_g, bn_b,
           W_e2, b_e2, W_e3, b_e3):
    src, dst = edge_index[0], edge_index[1]
    h = _tc_enc(x, W_enc, b_enc)
    for (wl, bl, wr, g, b) in (
        (W_l0, b_l0, W_r0, ln_g0, ln_b0),
        (W_l1, b_l1, W_r1, ln_g1, ln_b1),
        (W_l2, b_l2, W_r2, ln_g2, ln_b2),
    ):
        s, cnt, mx = _sc_agg(h, src, dst)
        h = _tc_layer(h, s[:N], cnt[:N].reshape(N, 1), mx[:N], wl, bl, wr, g, b)
    hs = _sc_gather(h, query_edge_indices[0])
    ht = _sc_gather(h, query_edge_indices[1])
    g_eff = bn_g / jnp.sqrt(1.0 + 1e-5)
    return _tc_mlp(hs, ht, edge_attr, W_e1[:H], W_e1[H:2 * H], W_e1[2 * H:],
                   b_e1, g_eff, bn_b, W_e2, b_e2, W_e3, b_e3)


# speedup vs baseline: 1.3414x; 1.3414x over previous
import functools

import jax
import jax.numpy as jnp
from jax import lax
from jax.experimental import pallas as pl
from jax.experimental.pallas import tpu as pltpu
from jax.experimental.pallas import tpu_sc as plsc

N = 10000
E = 320000
Q = 320000
D = 128
H = 128

NC = 2
NS = 16
NW = NC * NS
RPW = 320
NPAD = NW * RPW
NR = 336
DUMMY = 320
C = 2000
NCHUNK = E // C

QPW = Q // NW
CG = 128
NGC = 79


def _mesh():
    return plsc.VectorSubcoreMesh(core_axis_name="c", subcore_axis_name="s")


def _agg_body(h_hbm, src_hbm, dst_hbm, sum_hbm, cnt_hbm, max_hbm,
              asum, amax, acnt, dstb, srcb, cidx, cld, rows, tshift, sem):
    wid = lax.axis_index("s") * NC + lax.axis_index("c")
    lo = wid * RPW
    hi = lo + RPW

    zero16 = jnp.zeros((16,), jnp.float32)
    ninf16 = jnp.full((16,), -jnp.inf, jnp.float32)

    @pl.loop(0, NR)
    def _init(r):
        for kk in range(8):
            sl = pl.ds(16 * kk, 16)
            asum[r, sl] = zero16
            amax[r, sl] = ninf16

    @pl.loop(0, NR // 16)
    def _initc(r):
        acnt[pl.ds(16 * r, 16)] = zero16

    tshift[pl.ds(0, 16)] = jnp.zeros((16,), jnp.int32)

    @pl.loop(0, NCHUNK)
    def _chunk(c):
        off = pl.multiple_of(c * C, 8)
        pltpu.sync_copy(src_hbm.at[pl.ds(off, C)], srcb)
        pltpu.sync_copy(dst_hbm.at[pl.ds(off, C)], dstb)

        @pl.loop(0, C // 16, init_carry=0)
        def _scan(i, n):
            d = dstb[pl.ds(16 * i, 16)]
            s = srcb[pl.ds(16 * i, 16)]
            m = (d >= lo) & (d < hi)
            x = jnp.where(m, 1, 0)
            for sh in (1, 2, 4, 8):
                tshift[pl.ds(16, 16)] = x
                x = x + tshift[pl.ds(16 - sh, 16)]
            posv = n + x - 1
            plsc.store_scatter(cidx, [posv], s, mask=m)
            plsc.store_scatter(cld, [posv], d - lo, mask=m)
            return n + x[15]

        n = _scan
        cidx[pl.ds(n, 16)] = jnp.zeros((16,), jnp.int32)
        cld[pl.ds(n, 16)] = jnp.full((16,), DUMMY, jnp.int32)
        nb = (n + 15) // 16

        @pl.loop(0, nb)
        def _gblk(g):
            gi = cidx[pl.ds(16 * g, 16)]
            pltpu.async_copy(h_hbm.at[gi], rows, sem).wait()
            ld16 = cld[pl.ds(16 * g, 16)]
            for j in range(16):
                ld = ld16[j]
                for kk in range(8):
                    sl = pl.ds(16 * kk, 16)
                    r = rows[j, sl]
                    asum[ld, sl] = asum[ld, sl] + r
                    amax[ld, sl] = jnp.maximum(amax[ld, sl], r)
            plsc.addupdate_scatter(acnt, [ld16], jnp.ones((16,), jnp.float32))

    lo8 = pl.multiple_of(lo, 8)
    pltpu.sync_copy(asum.at[pl.ds(0, RPW)], sum_hbm.at[pl.ds(lo8, RPW)])
    pltpu.sync_copy(amax.at[pl.ds(0, RPW)], max_hbm.at[pl.ds(lo8, RPW)])
    pltpu.sync_copy(acnt.at[pl.ds(0, RPW)], cnt_hbm.at[pl.ds(lo8, RPW)])


@jax.jit
def _sc_agg(h, src, dst):
    f32 = jnp.float32
    return pl.kernel(
        _agg_body,
        out_type=(
            jax.ShapeDtypeStruct((NPAD, H), f32),
            jax.ShapeDtypeStruct((NPAD,), f32),
            jax.ShapeDtypeStruct((NPAD, H), f32),
        ),
        mesh=_mesh(),
        scratch_types=[
            pltpu.VMEM((NR, H), f32),
            pltpu.VMEM((NR, H), f32),
            pltpu.VMEM((NR,), f32),
            pltpu.VMEM((C,), jnp.int32),
            pltpu.VMEM((C,), jnp.int32),
            pltpu.VMEM((C + 16,), jnp.int32),
            pltpu.VMEM((C + 16,), jnp.int32),
            pltpu.VMEM((16, H), f32),
            pltpu.VMEM((32,), jnp.int32),
            pltpu.SemaphoreType.DMA,
        ],
        compiler_params=pltpu.CompilerParams(needs_layout_passes=False),
    )(h, src, dst)


def _gather_body(tab_hbm, idx_hbm, out_hbm, idxb, rowsb, sem):
    wid = lax.axis_index("s") * NC + lax.axis_index("c")
    base = wid * QPW

    @pl.loop(0, NGC)
    def _t(t):
        off = jnp.minimum(base + t * CG, Q - CG)
        off = pl.multiple_of(off, 8)
        pltpu.sync_copy(idx_hbm.at[pl.ds(off, CG)], idxb)
        pltpu.async_copy(tab_hbm.at[idxb], rowsb, sem).wait()
        pltpu.sync_copy(rowsb, out_hbm.at[pl.ds(off, CG)])


@jax.jit
def _sc_gather(tab, idx):
    return pl.kernel(
        _gather_body,
        out_type=jax.ShapeDtypeStruct((Q, H), jnp.float32),
        mesh=_mesh(),
        scratch_types=[
            pltpu.VMEM((CG,), jnp.int32),
            pltpu.VMEM((CG, H), jnp.float32),
            pltpu.SemaphoreType.DMA,
        ],
        compiler_params=pltpu.CompilerParams(needs_layout_passes=False),
    )(tab, idx)


def _enc_body(x_ref, w_ref, b_ref, o_ref):
    o_ref[...] = (
        jnp.dot(x_ref[...], w_ref[...], preferred_element_type=jnp.float32)
        + b_ref[...]
    )


@jax.jit
def _tc_enc(x, w, b):
    BN = 1000
    return pl.pallas_call(
        _enc_body,
        grid=(N // BN,),
        in_specs=[
            pl.BlockSpec((BN, D), lambda i: (i, 0)),
            pl.BlockSpec((D, H), lambda i: (0, 0)),
            pl.BlockSpec((H,), lambda i: (0,)),
        ],
        out_specs=pl.BlockSpec((BN, H), lambda i: (i, 0)),
        out_shape=jax.ShapeDtypeStruct((N, H), jnp.float32),
    )(x, w, b)


def _layer_body(h_ref, s_ref, c_ref, m_ref, wl_ref, bl_ref, wr_ref, g_ref, b_ref, o_ref):
    h = h_ref[...]
    cnt = c_ref[...]
    mean = s_ref[...] / jnp.maximum(cnt, 1.0)
    mx = m_ref[...]
    mx = jnp.where(jnp.isfinite(mx), mx, 0.0)
    agg = jnp.concatenate([mean, mx], axis=1)
    out = (
        jnp.dot(agg, wl_ref[...], preferred_element_type=jnp.float32)
        + bl_ref[...]
        + jnp.dot(h, wr_ref[...], preferred_element_type=jnp.float32)
    )
    mu = jnp.mean(out, axis=-1, keepdims=True)
    v = jnp.mean((out - mu) ** 2, axis=-1, keepdims=True)
    out = (out - mu) / jnp.sqrt(v + 1e-5) * g_ref[...] + b_ref[...]
    o_ref[...] = jnp.maximum(out, 0.0) + h


@jax.jit
def _tc_layer(h, s, cnt2, mx, wl, bl, wr, g, b):
    BN = 1000
    return pl.pallas_call(
        _layer_body,
        grid=(N // BN,),
        in_specs=[
            pl.BlockSpec((BN, H), lambda i: (i, 0)),
            pl.BlockSpec((BN, H), lambda i: (i, 0)),
            pl.BlockSpec((BN, 1), lambda i: (i, 0)),
            pl.BlockSpec((BN, H), lambda i: (i, 0)),
            pl.BlockSpec((2 * H, H), lambda i: (0, 0)),
            pl.BlockSpec((H,), lambda i: (0,)),
            pl.BlockSpec((H, H), lambda i: (0, 0)),
            pl.BlockSpec((H,), lambda i: (0,)),
            pl.BlockSpec((H,), lambda i: (0,)),
        ],
        out_specs=pl.BlockSpec((BN, H), lambda i: (i, 0)),
        out_shape=jax.ShapeDtypeStruct((N, H), jnp.float32),
    )(h, s, cnt2, mx, wl, bl, wr, g, b)


def _mlp_body(hs_ref, ht_ref, a_ref, w1a_ref, w1b_ref, w1c_ref, b1_ref,
              g_ref, bb_ref, w2_ref, b2_ref, w3_ref, b3_ref, o_ref):
    z = (
        jnp.dot(hs_ref[...], w1a_ref[...], preferred_element_type=jnp.float32)
        + jnp.dot(ht_ref[...], w1b_ref[...], preferred_element_type=jnp.float32)
        + jnp.dot(a_ref[...], w1c_ref[...], preferred_element_type=jnp.float32)
        + b1_ref[...]
    )
    z = z * g_ref[...] + bb_ref[...]
    z = jnp.maximum(z, 0.0)
    z = jnp.maximum(
        jnp.dot(z, w2_ref[...], preferred_element_type=jnp.float32) + b2_ref[...],
        0.0,
    )
    o_ref[...] = (
        jnp.dot(z, w3_ref[...], preferred_element_type=jnp.float32) + b3_ref[...]
    )


@jax.jit
def _tc_mlp(hs, ht, attr, w1a, w1b, w1c, b1, g, bb, w2, b2, w3, b3):
    BQ = 3200
    DE = attr.shape[1]
    return pl.pallas_call(
        _mlp_body,
        grid=(Q // BQ,),
        in_specs=[
            pl.BlockSpec((BQ, H), lambda i: (i, 0)),
            pl.BlockSpec((BQ, H), lambda i: (i, 0)),
            pl.BlockSpec((BQ, DE), lambda i: (i, 0)),
            pl.BlockSpec((H, 2 * H), lambda i: (0, 0)),
            pl.BlockSpec((H, 2 * H), lambda i: (0, 0)),
            pl.BlockSpec((DE, 2 * H), lambda i: (0, 0)),
            pl.BlockSpec((2 * H,), lambda i: (0,)),
            pl.BlockSpec((2 * H,), lambda i: (0,)),
            pl.BlockSpec((2 * H,), lambda i: (0,)),
            pl.BlockSpec((2 * H, H), lambda i: (0, 0)),
            pl.BlockSpec((H,), lambda i: (0,)),
            pl.BlockSpec((H, 1), lambda i: (0, 0)),
            pl.BlockSpec((1,), lambda i: (0,)),
        ],
        out_specs=pl.BlockSpec((BQ, 1), lambda i: (i, 0)),
        out_shape=jax.ShapeDtypeStruct((Q, 1), jnp.float32),
    )(hs, ht, attr, w1a, w1b, w1c, b1, g, bb, w2, b2, w3, b3)


def kernel(x, edge_index, edge_attr, query_edge_indices, W_enc, b_enc,
           W_l0, b_l0, W_r0, ln_g0, ln_b0, W_l1, b_l1, W_r1, ln_g1, ln_b1,
           W_l2, b_l2, W_r2, ln_g2, ln_b2, W_e1, b_e1, bn_g, bn_b,
           W_e2, b_e2, W_e3, b_e3):
    src, dst = edge_index[0], edge_index[1]
    h = _tc_enc(x, W_enc, b_enc)
    for (wl, bl, wr, g, b) in (
        (W_l0, b_l0, W_r0, ln_g0, ln_b0),
        (W_l1, b_l1, W_r1, ln_g1, ln_b1),
        (W_l2, b_l2, W_r2, ln_g2, ln_b2),
    ):
        s, cnt, mx = _sc_agg(h, src, dst)
        h = _tc_layer(h, s[:N], cnt[:N].reshape(N, 1), mx[:N], wl, bl, wr, g, b)
    hs = _sc_gather(h, query_edge_indices[0])
    ht = _sc_gather(h, query_edge_indices[1])
    g_eff = bn_g / jnp.sqrt(1.0 + 1e-5)
    return _tc_mlp(hs, ht, edge_attr, W_e1[:H], W_e1[H:2 * H], W_e1[2 * H:],
                   b_e1, g_eff, bn_b, W_e2, b_e2, W_e3, b_e3)


# speedup vs baseline: 2.2964x; 1.7120x over previous
import functools

import jax
import jax.numpy as jnp
from jax import lax
from jax.experimental import pallas as pl
from jax.experimental.pallas import tpu as pltpu
from jax.experimental.pallas import tpu_sc as plsc

N = 10000
E = 320000
Q = 320000
D = 128
H = 128

NC = 2
NS = 16
NW = NC * NS
RPW = 320
NPAD = NW * RPW
NR = 336
DUMMY = 320
C = 2000
NCHUNK = E // C

QPW = Q // NW
CG = 128
NGC = 79


def _mesh():
    return plsc.VectorSubcoreMesh(core_axis_name="c", subcore_axis_name="s")


F = 2048
EPW = 158 * F
SB = 2 * F + 16


def _part_body(src_hbm, dst_hbm, lists_hbm, blk_hbm, srcb, dstb, sbuf, cbuf):
    wid = lax.axis_index("s") * NC + lax.axis_index("c")
    lo = wid * RPW
    hi = lo + RPW
    iota16 = lax.broadcasted_iota(jnp.int32, (16,), 0)

    def _chunk(c, carry):
        p, fp = carry
        off = pl.multiple_of(c * C, 8)
        pltpu.sync_copy(src_hbm.at[pl.ds(off, C)], srcb)
        pltpu.sync_copy(dst_hbm.at[pl.ds(off, C)], dstb)

        def _scan(i, pn):
            sl = pl.ds(16 * i, 16)
            d = dstb[sl]
            m = (d >= lo) & (d < hi)

            def _do(pp):
                s = srcb[sl]
                packed = (s << 9) | (d - lo)
                sk, _, _ = plsc.sort_key_val(packed, packed, mask=m)
                sbuf[pl.ds(pp, 16)] = sk
                return pp + plsc.all_reduce_population_count(m)[0]

            return lax.cond(jnp.any(m), _do, lambda pp: pp, pn)

        p = pl.loop(0, C // 16, init_carry=p)(_scan)

        def _flush(args):
            pq, fq = args
            pltpu.sync_copy(sbuf.at[pl.ds(0, F)],
                            lists_hbm.at[pl.ds(pl.multiple_of(wid * EPW + fq, 8), F)])
            rem = pq - F

            @pl.loop(0, (rem + 15) // 16)
            def _move(k):
                sbuf[pl.ds(16 * k, 16)] = sbuf[pl.ds(F + 16 * k, 16)]

            return rem, fq + F

        return lax.cond(p >= F, _flush, lambda a: a, (p, fp))

    p, fp = pl.loop(0, NCHUNK, init_carry=(0, 0))(_chunk)

    sbuf[pl.ds(p, 16)] = jnp.full((16,), DUMMY, jnp.int32)
    nblk = (fp + p) // 16 + 1
    pltpu.sync_copy(sbuf.at[pl.ds(0, F)],
                    lists_hbm.at[pl.ds(pl.multiple_of(wid * EPW + fp, 8), F)])

    @pl.when(p + 16 > F)
    def _tail():
        pltpu.sync_copy(sbuf.at[pl.ds(F, F)],
                        lists_hbm.at[pl.ds(pl.multiple_of(wid * EPW + fp + F, 8), F)])

    cbuf[pl.ds(0, 16)] = jnp.where(iota16 == 0, nblk, 0)
    pltpu.sync_copy(cbuf, blk_hbm.at[pl.ds(pl.multiple_of(wid * 16, 8), 16)])


@jax.jit
def _sc_partition(src, dst):
    return pl.kernel(
        _part_body,
        out_type=(
            jax.ShapeDtypeStruct((NW * EPW,), jnp.int32),
            jax.ShapeDtypeStruct((NW * 16,), jnp.int32),
        ),
        mesh=_mesh(),
        scratch_types=[
            pltpu.VMEM((C,), jnp.int32),
            pltpu.VMEM((C,), jnp.int32),
            pltpu.VMEM((SB,), jnp.int32),
            pltpu.VMEM((16,), jnp.int32),
        ],
        compiler_params=pltpu.CompilerParams(needs_layout_passes=False),
    )(src, dst)


def _acc_body(h_hbm, lists_hbm, blk_hbm, sum_hbm, cnt_hbm, max_hbm,
              asum, amax, acnt, lbuf, rows0, rows1, cbuf, sem0, sem1):
    wid = lax.axis_index("s") * NC + lax.axis_index("c")
    lo = wid * RPW

    zero16 = jnp.zeros((16,), jnp.float32)
    ninf16 = jnp.full((16,), -jnp.inf, jnp.float32)
    ones16 = jnp.ones((16,), jnp.float32)

    @pl.loop(0, NR)
    def _init(r):
        for kk in range(8):
            sl = pl.ds(16 * kk, 16)
            asum[r, sl] = zero16
            amax[r, sl] = ninf16

    @pl.loop(0, NR // 16)
    def _initc(r):
        acnt[pl.ds(16 * r, 16)] = zero16

    pltpu.sync_copy(blk_hbm.at[pl.ds(pl.multiple_of(wid * 16, 8), 16)], cbuf)
    nblk = cbuf[pl.ds(0, 16)][0]
    nck = (nblk + 127) // 128

    def _accum(v, rows):
        ld16 = v & 511
        for j in range(16):
            ld = ld16[j]
            for kk in range(8):
                sl = pl.ds(16 * kk, 16)
                r = rows[j, sl]
                asum[ld, sl] = asum[ld, sl] + r
                amax[ld, sl] = jnp.maximum(amax[ld, sl], r)
        plsc.addupdate_scatter(acnt, [ld16], ones16)

    @pl.loop(0, nck)
    def _chunk(c):
        pltpu.sync_copy(lists_hbm.at[pl.ds(pl.multiple_of(wid * EPW + c * F, 8), F)],
                        lbuf)
        bic = jnp.minimum(nblk - c * 128, 128)

        @pl.loop(0, (bic + 1) // 2)
        def _pair(q):
            b1 = 2 * q + 1
            have1 = b1 < bic
            v0 = lbuf[pl.ds(32 * q, 16)]
            pltpu.async_copy(h_hbm.at[(v0 >> 9)], rows0, sem0)

            @pl.when(have1)
            def _issue1():
                v1 = lbuf[pl.ds(32 * q + 16, 16)]
                pltpu.async_copy(h_hbm.at[(v1 >> 9)], rows1, sem1)

            pltpu.make_async_copy(h_hbm.at[pl.ds(0, 16)], rows0, sem0).wait()
            _accum(v0, rows0)

            @pl.when(have1)
            def _acc1():
                pltpu.make_async_copy(h_hbm.at[pl.ds(0, 16)], rows1, sem1).wait()
                _accum(lbuf[pl.ds(32 * q + 16, 16)], rows1)

    lo8 = pl.multiple_of(lo, 8)
    pltpu.sync_copy(asum.at[pl.ds(0, RPW)], sum_hbm.at[pl.ds(lo8, RPW)])
    pltpu.sync_copy(amax.at[pl.ds(0, RPW)], max_hbm.at[pl.ds(lo8, RPW)])
    pltpu.sync_copy(acnt.at[pl.ds(0, RPW)], cnt_hbm.at[pl.ds(lo8, RPW)])


@jax.jit
def _sc_accum(h, lists, blk):
    f32 = jnp.float32
    return pl.kernel(
        _acc_body,
        out_type=(
            jax.ShapeDtypeStruct((NPAD, H), f32),
            jax.ShapeDtypeStruct((NPAD,), f32),
            jax.ShapeDtypeStruct((NPAD, H), f32),
        ),
        mesh=_mesh(),
        scratch_types=[
            pltpu.VMEM((NR, H), f32),
            pltpu.VMEM((NR, H), f32),
            pltpu.VMEM((NR,), f32),
            pltpu.VMEM((F,), jnp.int32),
            pltpu.VMEM((16, H), f32),
            pltpu.VMEM((16, H), f32),
            pltpu.VMEM((16,), jnp.int32),
            pltpu.SemaphoreType.DMA,
            pltpu.SemaphoreType.DMA,
        ],
        compiler_params=pltpu.CompilerParams(needs_layout_passes=False),
    )(h, lists, blk)


def _gather_body(tab_hbm, idx_hbm, out_hbm, idxb, rowsb, sem):
    wid = lax.axis_index("s") * NC + lax.axis_index("c")
    base = wid * QPW

    @pl.loop(0, NGC)
    def _t(t):
        off = jnp.minimum(base + t * CG, Q - CG)
        off = pl.multiple_of(off, 8)
        pltpu.sync_copy(idx_hbm.at[pl.ds(off, CG)], idxb)
        pltpu.async_copy(tab_hbm.at[idxb], rowsb, sem).wait()
        pltpu.sync_copy(rowsb, out_hbm.at[pl.ds(off, CG)])


@jax.jit
def _sc_gather(tab, idx):
    return pl.kernel(
        _gather_body,
        out_type=jax.ShapeDtypeStruct((Q, H), jnp.float32),
        mesh=_mesh(),
        scratch_types=[
            pltpu.VMEM((CG,), jnp.int32),
            pltpu.VMEM((CG, H), jnp.float32),
            pltpu.SemaphoreType.DMA,
        ],
        compiler_params=pltpu.CompilerParams(needs_layout_passes=False),
    )(tab, idx)


def _enc_body(x_ref, w_ref, b_ref, o_ref):
    o_ref[...] = (
        jnp.dot(x_ref[...], w_ref[...], preferred_element_type=jnp.float32)
        + b_ref[...]
    )


@jax.jit
def _tc_enc(x, w, b):
    BN = 1000
    return pl.pallas_call(
        _enc_body,
        grid=(N // BN,),
        in_specs=[
            pl.BlockSpec((BN, D), lambda i: (i, 0)),
            pl.BlockSpec((D, H), lambda i: (0, 0)),
            pl.BlockSpec((H,), lambda i: (0,)),
        ],
        out_specs=pl.BlockSpec((BN, H), lambda i: (i, 0)),
        out_shape=jax.ShapeDtypeStruct((N, H), jnp.float32),
    )(x, w, b)


def _layer_body(h_ref, s_ref, c_ref, m_ref, wl_ref, bl_ref, wr_ref, g_ref, b_ref, o_ref):
    h = h_ref[...]
    cnt = c_ref[...]
    mean = s_ref[...] / jnp.maximum(cnt, 1.0)
    mx = m_ref[...]
    mx = jnp.where(jnp.isfinite(mx), mx, 0.0)
    agg = jnp.concatenate([mean, mx], axis=1)
    out = (
        jnp.dot(agg, wl_ref[...], preferred_element_type=jnp.float32)
        + bl_ref[...]
        + jnp.dot(h, wr_ref[...], preferred_element_type=jnp.float32)
    )
    mu = jnp.mean(out, axis=-1, keepdims=True)
    v = jnp.mean((out - mu) ** 2, axis=-1, keepdims=True)
    out = (out - mu) / jnp.sqrt(v + 1e-5) * g_ref[...] + b_ref[...]
    o_ref[...] = jnp.maximum(out, 0.0) + h


@jax.jit
def _tc_layer(h, s, cnt2, mx, wl, bl, wr, g, b):
    BN = 1000
    return pl.pallas_call(
        _layer_body,
        grid=(N // BN,),
        in_specs=[
            pl.BlockSpec((BN, H), lambda i: (i, 0)),
            pl.BlockSpec((BN, H), lambda i: (i, 0)),
            pl.BlockSpec((BN, 1), lambda i: (i, 0)),
            pl.BlockSpec((BN, H), lambda i: (i, 0)),
            pl.BlockSpec((2 * H, H), lambda i: (0, 0)),
            pl.BlockSpec((H,), lambda i: (0,)),
            pl.BlockSpec((H, H), lambda i: (0, 0)),
            pl.BlockSpec((H,), lambda i: (0,)),
            pl.BlockSpec((H,), lambda i: (0,)),
        ],
        out_specs=pl.BlockSpec((BN, H), lambda i: (i, 0)),
        out_shape=jax.ShapeDtypeStruct((N, H), jnp.float32),
    )(h, s, cnt2, mx, wl, bl, wr, g, b)


def _mlp_body(hs_ref, ht_ref, a_ref, w1a_ref, w1b_ref, w1c_ref, b1_ref,
              g_ref, bb_ref, w2_ref, b2_ref, w3_ref, b3_ref, o_ref):
    z = (
        jnp.dot(hs_ref[...], w1a_ref[...], preferred_element_type=jnp.float32)
        + jnp.dot(ht_ref[...], w1b_ref[...], preferred_element_type=jnp.float32)
        + jnp.dot(a_ref[...], w1c_ref[...], preferred_element_type=jnp.float32)
        + b1_ref[...]
    )
    z = z * g_ref[...] + bb_ref[...]
    z = jnp.maximum(z, 0.0)
    z = jnp.maximum(
        jnp.dot(z, w2_ref[...], preferred_element_type=jnp.float32) + b2_ref[...],
        0.0,
    )
    o_ref[...] = (
        jnp.dot(z, w3_ref[...], preferred_element_type=jnp.float32) + b3_ref[...]
    )


@jax.jit
def _tc_mlp(hs, ht, attr, w1a, w1b, w1c, b1, g, bb, w2, b2, w3, b3):
    BQ = 3200
    DE = attr.shape[1]
    return pl.pallas_call(
        _mlp_body,
        grid=(Q // BQ,),
        in_specs=[
            pl.BlockSpec((BQ, H), lambda i: (i, 0)),
            pl.BlockSpec((BQ, H), lambda i: (i, 0)),
            pl.BlockSpec((BQ, DE), lambda i: (i, 0)),
            pl.BlockSpec((H, 2 * H), lambda i: (0, 0)),
            pl.BlockSpec((H, 2 * H), lambda i: (0, 0)),
            pl.BlockSpec((DE, 2 * H), lambda i: (0, 0)),
            pl.BlockSpec((2 * H,), lambda i: (0,)),
            pl.BlockSpec((2 * H,), lambda i: (0,)),
            pl.BlockSpec((2 * H,), lambda i: (0,)),
            pl.BlockSpec((2 * H, H), lambda i: (0, 0)),
            pl.BlockSpec((H,), lambda i: (0,)),
            pl.BlockSpec((H, 1), lambda i: (0, 0)),
            pl.BlockSpec((1,), lambda i: (0,)),
        ],
        out_specs=pl.BlockSpec((BQ, 1), lambda i: (i, 0)),
        out_shape=jax.ShapeDtypeStruct((Q, 1), jnp.float32),
    )(hs, ht, attr, w1a, w1b, w1c, b1, g, bb, w2, b2, w3, b3)


def kernel(x, edge_index, edge_attr, query_edge_indices, W_enc, b_enc,
           W_l0, b_l0, W_r0, ln_g0, ln_b0, W_l1, b_l1, W_r1, ln_g1, ln_b1,
           W_l2, b_l2, W_r2, ln_g2, ln_b2, W_e1, b_e1, bn_g, bn_b,
           W_e2, b_e2, W_e3, b_e3):
    src, dst = edge_index[0], edge_index[1]
    h = _tc_enc(x, W_enc, b_enc)
    lists, blk = _sc_partition(src, dst)
    for (wl, bl, wr, g, b) in (
        (W_l0, b_l0, W_r0, ln_g0, ln_b0),
        (W_l1, b_l1, W_r1, ln_g1, ln_b1),
        (W_l2, b_l2, W_r2, ln_g2, ln_b2),
    ):
        s, cnt, mx = _sc_accum(h, lists, blk)
        h = _tc_layer(h, s[:N], cnt[:N].reshape(N, 1), mx[:N], wl, bl, wr, g, b)
    hs = _sc_gather(h, query_edge_indices[0])
    ht = _sc_gather(h, query_edge_indices[1])
    g_eff = bn_g / jnp.sqrt(1.0 + 1e-5)
    return _tc_mlp(hs, ht, edge_attr, W_e1[:H], W_e1[H:2 * H], W_e1[2 * H:],
                   b_e1, g_eff, bn_b, W_e2, b_e2, W_e3, b_e3)
